# baseline (speedup 1.0000x reference)
import jax
import jax.numpy as jnp
from jax.experimental import pallas as pl
from jax.experimental.pallas import tpu as pltpu

CUT_ = 64
D_ = 8


def _body(xv_ref, ts_ref, y_ref, o_ref):
    bb = y_ref.shape[0]
    k = y_ref.shape[3]
    xb = jnp.broadcast_to(xv_ref[...], (bb, 32, 8, k))
    o_ref[:, :32, :8, :] = xb
    o_ref[:, 32:, :8, :] = y_ref[...]
    o_ref[:, :, 8:9, :] = jnp.broadcast_to(ts_ref[...], (bb, 64, 1, k))


def kernel(x, y):
    squeeze = False
    if y.ndim == 2:
        y = y[:, :, None]
        squeeze = True
    B = x.shape[0]
    K = y.shape[2]
    C = CUT_
    NV = (x.shape[1] - C) // D_
    NY = y.shape[1] // D_

    xv = x[:, C:].reshape(B, NV, D_, 1)
    ts = x[:, :C].reshape(B, C, 1, 1)
    yr = y.reshape(B, NY, D_, K)

    Bb = 16
    grid = (B // Bb,)

    out = pl.pallas_call(
        _body,
        grid=grid,
        in_specs=[
            pl.BlockSpec((Bb, NV, D_, 1), lambda i: (i, 0, 0, 0)),
            pl.BlockSpec((Bb, C, 1, 1), lambda i: (i, 0, 0, 0)),
            pl.BlockSpec((Bb, NY, D_, K), lambda i: (i, 0, 0, 0)),
        ],
        out_specs=pl.BlockSpec((Bb, C, D_ + 1, K), lambda i: (i, 0, 0, 0)),
        out_shape=jax.ShapeDtypeStruct((B, C, D_ + 1, K), x.dtype),
        compiler_params=pltpu.CompilerParams(
            dimension_semantics=("arbitrary",),
        ),
    )(xv, ts, yr)

    if squeeze:
        out = jnp.squeeze(out, axis=-1)
    return out
